# baseline (speedup 1.0000x reference)
import functools
import math

import jax
import jax.numpy as jnp
from jax import lax
from jax.experimental import pallas as pl
from jax.experimental.pallas import tpu as pltpu
from jax.experimental.pallas import tpu_sc as plsc

_CH = 256
_NBUF = 2


@functools.lru_cache(maxsize=None)
def _make_gather(V, D, N, S):
  mesh = plsc.VectorSubcoreMesh(core_axis_name="c", subcore_axis_name="s")
  NC, NS = mesh.num_cores, mesh.num_subcores
  NW = NC * NS
  assert D % 8 == 0 and N % (NW * _CH) == 0 and _CH % 128 == 0
  JB = D // 8
  NBLK = N // 128
  CPW = N // (NW * _CH)
  nvisit = S * CPW
  assert nvisit % _NBUF == 0
  nb_per_ch = _CH // 128
  scale = math.sqrt(D)

  @functools.partial(
      pl.kernel,
      out_type=jax.ShapeDtypeStruct((S, JB, NBLK, 8 * 128), jnp.float32),
      mesh=mesh,
      scratch_types=[
          pltpu.VMEM((S, CPW * _CH), jnp.int32),
          pltpu.VMEM((_NBUF, _CH, D), jnp.float32),
          pltpu.VMEM((_NBUF, _CH * D), jnp.float32),
          [pltpu.SemaphoreType.DMA] * _NBUF,
          [pltpu.SemaphoreType.DMA] * _NBUF,
      ],
      compiler_params=pltpu.CompilerParams(
          use_tc_tiling_on_sc=False, needs_layout_passes=False),
  )
  def gather_kernel(tokt_hbm, table_hbm, out_hbm, idx_v, rin, rout, gsems, osems):
    wid = lax.axis_index("s") * NC + lax.axis_index("c")
    nbase = wid * (CPW * _CH)

    pltpu.sync_copy(tokt_hbm.at[:, pl.ds(nbase, CPW * _CH)], idx_v)

    def visit_sc(t):
      s = t // CPW
      cc = t - s * CPW
      return s, cc

    def fire_gather(t, b):
      s, cc = visit_sc(t)
      for j in range(_CH // 128):
        pltpu.make_async_copy(
            table_hbm.at[idx_v.at[s, pl.ds(cc * _CH + j * 128, 128)]],
            rin.at[b, pl.ds(j * 128, 128), :],
            gsems[b]).start()

    def wait_gather(t, b):
      s, cc = visit_sc(t)
      for j in range(_CH // 128):
        pltpu.make_async_copy(
            table_hbm.at[idx_v.at[s, pl.ds(cc * _CH + j * 128, 128)]],
            rin.at[b, pl.ds(j * 128, 128), :],
            gsems[b]).wait()

    def _write_copies(t, b):
      s, cc = visit_sc(t)
      nb0 = wid * (CPW * nb_per_ch) + cc * nb_per_ch
      return [
          pltpu.make_async_copy(
              rout.at[b, pl.ds((jb * nb_per_ch + k) * 1024, 1024)],
              out_hbm.at[s, jb, nb0 + k, :], osems[b])
          for jb in range(JB) for k in range(nb_per_ch)
      ]

    def fire_write(t, b):
      for c in _write_copies(t, b):
        c.start()

    def wait_write(t, b):
      for c in _write_copies(t, b):
        c.wait()

    for b in range(_NBUF):
      fire_gather(b, b)

    iota16 = lax.iota(jnp.int32, 16)
    zero16 = iota16 * 0
    srcv_c = []
    dstv_c = []
    for d in range(16):
      e = (iota16 + d) & 15
      srcv_c.append(iota16 * D + e)
      dstv_c.append(((((e >> 3) << 4) + (e & 7)) << 7) + iota16)
    PN = _CH // 16
    QN = D // 16

    @pl.loop(0, nvisit, step=_NBUF)
    def _outer(t0):
      for b in range(_NBUF):
        t = t0 + b
        wait_gather(t, b)

        @pl.when(t0 > 0)
        def _():
          wait_write(t - _NBUF, b)

        rin_b = rin.at[b]
        rout_b = rout.at[b]

        @plsc.parallel_loop(0, PN, 1, unroll=2)
        def _tr(p):
          sp = p * (16 * D)
          dp = ((p >> 3) << 10) + ((p & 7) << 4)
          for q in range(QN):
            so = sp + q * 16
            do = dp + q * 4096
            for d0 in range(0, 16, 4):
              vals = [plsc.load_gather(rin_b, [zero16, srcv_c[d0 + i] + so])
                      * scale for i in range(4)]
              for i in range(4):
                plsc.store_scatter(rout_b, [dstv_c[d0 + i] + do], vals[i])

        @pl.when(t + _NBUF < nvisit)
        def _():
          fire_gather(t + _NBUF, b)

        fire_write(t, b)

    for b in range(_NBUF):
      wait_write(nvisit - _NBUF + b, b)

  return gather_kernel


def kernel(tokens, table):
  n, s = tokens.shape
  V, D = table.shape
  gather_fn = _make_gather(V, D, n, s)
  tokt = tokens.T.astype(jnp.int32)
  out4 = gather_fn(tokt, table)
  out5 = out4.reshape(s, D // 8, n // 128, 8, 128)
  return out5.transpose(2, 4, 0, 1, 3).reshape(n, s, D)


# speedup vs baseline: 1.2809x; 1.2809x over previous
import functools
import math

import jax
import jax.numpy as jnp
from jax import lax
from jax.experimental import pallas as pl
from jax.experimental.pallas import tpu as pltpu
from jax.experimental.pallas import tpu_sc as plsc

_CH = 256
_NBUF = 2


@functools.lru_cache(maxsize=None)
def _make_gather(V, D, N, S):
  mesh = plsc.VectorSubcoreMesh(core_axis_name="c", subcore_axis_name="s")
  NC, NS = mesh.num_cores, mesh.num_subcores
  NW = NC * NS
  assert D % 8 == 0 and N % (NW * _CH) == 0 and _CH % 128 == 0
  JB = D // 8
  NBLK = N // 128
  CPW = N // (NW * _CH)
  nvisit = S * CPW
  assert nvisit % _NBUF == 0
  nb_per_ch = _CH // 128
  scale = math.sqrt(D)

  @functools.partial(
      pl.kernel,
      out_type=jax.ShapeDtypeStruct((S, JB, NBLK, 8, 128), jnp.float32),
      mesh=mesh,
      scratch_types=[
          pltpu.VMEM((S, CPW * _CH), jnp.int32),
          pltpu.VMEM((_NBUF, _CH, D), jnp.float32),
          pltpu.VMEM((_NBUF, JB * nb_per_ch * 8, 128), jnp.float32),
          [pltpu.SemaphoreType.DMA] * _NBUF,
          [pltpu.SemaphoreType.DMA] * _NBUF,
      ],
      compiler_params=pltpu.CompilerParams(
          use_tc_tiling_on_sc=False, needs_layout_passes=False),
  )
  def gather_kernel(tokt_hbm, table_hbm, out_hbm, idx_v, rin, rout, gsems, osems):
    wid = lax.axis_index("s") * NC + lax.axis_index("c")
    nbase = wid * (CPW * _CH)

    pltpu.sync_copy(tokt_hbm.at[:, pl.ds(nbase, CPW * _CH)], idx_v)

    def visit_sc(t):
      s = t // CPW
      cc = t - s * CPW
      return s, cc

    def fire_gather(t, b):
      s, cc = visit_sc(t)
      for j in range(_CH // 128):
        pltpu.make_async_copy(
            table_hbm.at[idx_v.at[s, pl.ds(cc * _CH + j * 128, 128)]],
            rin.at[b, pl.ds(j * 128, 128), :],
            gsems[b]).start()

    def wait_gather(t, b):
      s, cc = visit_sc(t)
      for j in range(_CH // 128):
        pltpu.make_async_copy(
            table_hbm.at[idx_v.at[s, pl.ds(cc * _CH + j * 128, 128)]],
            rin.at[b, pl.ds(j * 128, 128), :],
            gsems[b]).wait()

    def _write_copies(t, b):
      s, cc = visit_sc(t)
      nb0 = wid * (CPW * nb_per_ch) + cc * nb_per_ch
      return [
          pltpu.make_async_copy(
              rout.at[b, pl.ds((jb * nb_per_ch + k) * 8, 8), :],
              out_hbm.at[s, jb, nb0 + k], osems[b])
          for jb in range(JB) for k in range(nb_per_ch)
      ]

    def fire_write(t, b):
      for c in _write_copies(t, b):
        c.start()

    def wait_write(t, b):
      for c in _write_copies(t, b):
        c.wait()

    for b in range(_NBUF):
      fire_gather(b, b)

    iota16 = lax.iota(jnp.int32, 16)
    cols_c = []
    rowv_c = []
    for d in range(16):
      e = (iota16 + d) & 15
      cols_c.append(e)
      rowv_c.append(((e >> 3) << 4) + (e & 7))
    PN = _CH // 16
    QN = D // 16

    @pl.loop(0, nvisit, step=_NBUF)
    def _outer(t0):
      for b in range(_NBUF):
        t = t0 + b
        wait_gather(t, b)

        @pl.when(t0 > 0)
        def _():
          wait_write(t - _NBUF, b)

        rin_b = rin.at[b]
        rout_b = rout.at[b]

        @plsc.parallel_loop(0, PN, 1, unroll=4)
        def _tr(p):
          rows = iota16 + p * 16
          colv = iota16 + (p & 7) * 16
          rbase = (p >> 3) * 8
          for q in range(QN):
            cq = q * 16
            rq = rbase + q * 32
            for d0 in range(0, 16, 4):
              vals = [plsc.load_gather(rin_b, [rows, cols_c[d0 + i] + cq])
                      * scale for i in range(4)]
              for i in range(4):
                plsc.store_scatter(
                    rout_b, [rowv_c[d0 + i] + rq, colv], vals[i])

        @pl.when(t + _NBUF < nvisit)
        def _():
          fire_gather(t + _NBUF, b)

        fire_write(t, b)

    for b in range(_NBUF):
      wait_write(nvisit - _NBUF + b, b)

  return gather_kernel


def kernel(tokens, table):
  n, s = tokens.shape
  V, D = table.shape
  gather_fn = _make_gather(V, D, n, s)
  tokt = tokens.T.astype(jnp.int32)
  out5 = gather_fn(tokt, table)
  return out5.transpose(2, 4, 0, 1, 3).reshape(n, s, D)


# speedup vs baseline: 1.3936x; 1.0880x over previous
import functools
import math

import jax
import jax.numpy as jnp
from jax import lax
from jax.experimental import pallas as pl
from jax.experimental.pallas import tpu as pltpu
from jax.experimental.pallas import tpu_sc as plsc

_CH = 256
_NBUF = 2


@functools.lru_cache(maxsize=None)
def _make_gather(V, D, N, S):
  mesh = plsc.VectorSubcoreMesh(core_axis_name="c", subcore_axis_name="s")
  NC, NS = mesh.num_cores, mesh.num_subcores
  NW = NC * NS
  assert D % 8 == 0 and N % (NW * _CH) == 0 and _CH % 128 == 0
  JB = D // 8
  NBLK = N // 128
  CPW = N // (NW * _CH)
  nvisit = S * CPW
  assert nvisit % _NBUF == 0
  nb_per_ch = _CH // 128
  scale = math.sqrt(D)

  @functools.partial(
      pl.kernel,
      out_type=jax.ShapeDtypeStruct((S, JB, NBLK, 8, 128), jnp.float32),
      mesh=mesh,
      scratch_types=[
          pltpu.VMEM((S, CPW * _CH), jnp.int32),
          pltpu.VMEM((_NBUF, _CH, D), jnp.float32),
          pltpu.VMEM((_NBUF, JB * nb_per_ch * 8, 128), jnp.float32),
          [pltpu.SemaphoreType.DMA] * _NBUF,
          [pltpu.SemaphoreType.DMA] * _NBUF,
      ],
      compiler_params=pltpu.CompilerParams(
          use_tc_tiling_on_sc=False, needs_layout_passes=False),
  )
  def gather_kernel(tokt_hbm, table_hbm, out_hbm, idx_v, rin, rout, gsems, osems):
    wid = lax.axis_index("s") * NC + lax.axis_index("c")
    nbase = wid * (CPW * _CH)

    pltpu.sync_copy(tokt_hbm.at[:, pl.ds(nbase, CPW * _CH)], idx_v)

    def visit_sc(t):
      s = t // CPW
      cc = t - s * CPW
      return s, cc

    def fire_gather(t, b):
      s, cc = visit_sc(t)
      for j in range(_CH // 128):
        pltpu.make_async_copy(
            table_hbm.at[idx_v.at[s, pl.ds(cc * _CH + j * 128, 128)]],
            rin.at[b, pl.ds(j * 128, 128), :],
            gsems[b]).start()

    def wait_gather(t, b):
      s, cc = visit_sc(t)
      for j in range(_CH // 128):
        pltpu.make_async_copy(
            table_hbm.at[idx_v.at[s, pl.ds(cc * _CH + j * 128, 128)]],
            rin.at[b, pl.ds(j * 128, 128), :],
            gsems[b]).wait()

    def _write_copies(t, b):
      s, cc = visit_sc(t)
      nb0 = wid * (CPW * nb_per_ch) + cc * nb_per_ch
      return [
          pltpu.make_async_copy(
              rout.at[b, pl.ds((jb * nb_per_ch + k) * 8, 8), :],
              out_hbm.at[s, jb, nb0 + k], osems[b])
          for jb in range(JB) for k in range(nb_per_ch)
      ]

    def fire_write(t, b):
      for c in _write_copies(t, b):
        c.start()

    def wait_write(t, b):
      for c in _write_copies(t, b):
        c.wait()

    for b in range(_NBUF):
      fire_gather(b, b)

    iota16 = lax.iota(jnp.int32, 16)
    zero16 = iota16 * 0
    srcv_c = []
    rowv_c = []
    for d in range(16):
      e = (iota16 + d) & 15
      srcv_c.append(iota16 * D + e)
      rowv_c.append(((e >> 3) << 4) + (e & 7))
    PN = _CH // 16
    QN = D // 16

    @pl.loop(0, nvisit, step=_NBUF)
    def _outer(t0):
      for b in range(_NBUF):
        t = t0 + b
        wait_gather(t, b)

        @pl.when(t0 > 0)
        def _():
          wait_write(t - _NBUF, b)

        rin_b = rin.at[b]
        rout_b = rout.at[b]

        @plsc.parallel_loop(0, PN, 1, unroll=4)
        def _tr(p):
          sp = p * (16 * D)
          colv = iota16 + (p & 7) * 16
          rbase = (p >> 3) * 8
          for q in range(QN):
            so = sp + q * 16
            rq = rbase + q * 32
            for d0 in range(0, 16, 4):
              vals = [plsc.load_gather(rin_b, [zero16, srcv_c[d0 + i] + so])
                      * scale for i in range(4)]
              for i in range(4):
                plsc.store_scatter(
                    rout_b, [rowv_c[d0 + i] + rq, colv], vals[i])

        @pl.when(t + _NBUF < nvisit)
        def _():
          fire_gather(t + _NBUF, b)

        fire_write(t, b)

    for b in range(_NBUF):
      wait_write(nvisit - _NBUF + b, b)

  return gather_kernel


def kernel(tokens, table):
  n, s = tokens.shape
  V, D = table.shape
  gather_fn = _make_gather(V, D, n, s)
  tokt = tokens.T.astype(jnp.int32)
  out5 = gather_fn(tokt, table)
  return out5.transpose(2, 4, 0, 1, 3).reshape(n, s, D)


# speedup vs baseline: 1.4265x; 1.0236x over previous
import functools
import math

import jax
import jax.numpy as jnp
from jax import lax
from jax.experimental import pallas as pl
from jax.experimental.pallas import tpu as pltpu
from jax.experimental.pallas import tpu_sc as plsc

_CH = 256
_NBUF = 2


@functools.lru_cache(maxsize=None)
def _make_gather(V, D, N, S):
  mesh = plsc.VectorSubcoreMesh(core_axis_name="c", subcore_axis_name="s")
  NC, NS = mesh.num_cores, mesh.num_subcores
  NW = NC * NS
  assert D % 8 == 0 and N % (NW * _CH) == 0 and _CH % 128 == 0
  JB = D // 8
  NBLK = N // 128
  CPW = N // (NW * _CH)
  nvisit = S * CPW
  assert nvisit % _NBUF == 0
  nb_per_ch = _CH // 128
  scale = math.sqrt(D)

  @functools.partial(
      pl.kernel,
      out_type=jax.ShapeDtypeStruct((S, JB, NBLK, 8, 128), jnp.float32),
      mesh=mesh,
      scratch_types=[
          pltpu.VMEM((S, CPW * _CH), jnp.int32),
          pltpu.VMEM((_NBUF, _CH, D), jnp.float32),
          pltpu.VMEM((_NBUF, JB * nb_per_ch * 8, 128), jnp.float32),
          [pltpu.SemaphoreType.DMA] * _NBUF,
          [pltpu.SemaphoreType.DMA] * _NBUF,
      ],
      compiler_params=pltpu.CompilerParams(
          use_tc_tiling_on_sc=False, needs_layout_passes=False),
  )
  def gather_kernel(tokt_hbm, table_hbm, out_hbm, idx_v, rin, rout, gsems, osems):
    wid = lax.axis_index("s") * NC + lax.axis_index("c")
    nbase = wid * (CPW * _CH)

    pltpu.sync_copy(tokt_hbm.at[:, pl.ds(nbase, CPW * _CH)], idx_v)

    def visit_sc(t):
      s = t // CPW
      cc = t - s * CPW
      return s, cc

    def fire_gather(t, b):
      s, cc = visit_sc(t)
      for j in range(_CH // 128):
        pltpu.make_async_copy(
            table_hbm.at[idx_v.at[s, pl.ds(cc * _CH + j * 128, 128)]],
            rin.at[b, pl.ds(j * 128, 128), :],
            gsems[b]).start()

    def wait_gather(t, b):
      s, cc = visit_sc(t)
      for j in range(_CH // 128):
        pltpu.make_async_copy(
            table_hbm.at[idx_v.at[s, pl.ds(cc * _CH + j * 128, 128)]],
            rin.at[b, pl.ds(j * 128, 128), :],
            gsems[b]).wait()

    def _write_copies(t, b):
      s, cc = visit_sc(t)
      nb0 = wid * (CPW * nb_per_ch) + cc * nb_per_ch
      return [
          pltpu.make_async_copy(
              rout.at[b, pl.ds((jb * nb_per_ch + k) * 8, 8), :],
              out_hbm.at[s, jb, nb0 + k], osems[b])
          for jb in range(JB) for k in range(nb_per_ch)
      ]

    def fire_write(t, b):
      for c in _write_copies(t, b):
        c.start()

    def wait_write(t, b):
      for c in _write_copies(t, b):
        c.wait()

    for b in range(_NBUF):
      fire_gather(b, b)

    iota16 = lax.iota(jnp.int32, 16)
    zero16 = iota16 * 0
    srcv_c = []
    rowv_c = []
    for d in range(16):
      e = (iota16 + d) & 15
      srcv_c.append(iota16 * D + e)
      rowv_c.append(((((e >> 3) << 4) + (e & 7)) << 7) + iota16)
    PN = _CH // 16
    QN = D // 16

    @pl.loop(0, nvisit, step=_NBUF)
    def _outer(t0):
      for b in range(_NBUF):
        t = t0 + b
        wait_gather(t, b)

        @pl.when(t0 > 0)
        def _():
          wait_write(t - _NBUF, b)

        rin_b = rin.at[b]
        rout_b = rout.at[b]

        @plsc.parallel_loop(0, PN, 1, unroll=4)
        def _tr(p):
          sp = p * (16 * D)
          dbase = ((p >> 3) << 10) + ((p & 7) << 4)
          for q in range(QN):
            so = sp + q * 16
            do = dbase + q * 4096
            for d0 in range(0, 16, 4):
              vals = [plsc.load_gather(rin_b, [zero16, srcv_c[d0 + i] + so])
                      * scale for i in range(4)]
              for i in range(4):
                plsc.store_scatter(
                    rout_b, [zero16, rowv_c[d0 + i] + do], vals[i])

        @pl.when(t + _NBUF < nvisit)
        def _():
          fire_gather(t + _NBUF, b)

        fire_write(t, b)

    for b in range(_NBUF):
      wait_write(nvisit - _NBUF + b, b)

  return gather_kernel


def kernel(tokens, table):
  n, s = tokens.shape
  V, D = table.shape
  gather_fn = _make_gather(V, D, n, s)
  tokt = tokens.T.astype(jnp.int32)
  out5 = gather_fn(tokt, table)
  return out5.transpose(2, 4, 0, 1, 3).reshape(n, s, D)
